# baseline (speedup 1.0000x reference)
import functools

import jax
import jax.numpy as jnp
from jax import lax
from jax.experimental import pallas as pl
from jax.experimental.pallas import tpu as pltpu
from jax.experimental.pallas import tpu_sc as plsc

NUM_CORES = 2
NUM_SUBCORES = 16
NUM_WORKERS = NUM_CORES * NUM_SUBCORES
LANES = 16
KC = 8
PAR = 3


@jax.jit
def _pos_embed_add(x, table, pos2d):
    B, S, D = x.shape
    s_per_w = S // NUM_WORKERS
    n_chunks = s_per_w // KC

    @functools.partial(
        pl.kernel,
        out_type=jax.ShapeDtypeStruct((B, S, D), jnp.float32),
        mesh=plsc.VectorSubcoreMesh(
            core_axis_name="c", subcore_axis_name="s"
        ),
        scratch_types=[
            pltpu.VMEM((s_per_w,), jnp.int32),
            pltpu.VMEM((PAR, KC, D), jnp.float32),
            pltpu.VMEM((PAR * B, KC, D), jnp.float32),
            pltpu.SemaphoreType.DMA((PAR,)),
            pltpu.SemaphoreType.DMA((PAR,)),
            pltpu.SemaphoreType.DMA((PAR,)),
        ],
    )
    def body(x_hbm, table_hbm, pos_hbm, out_hbm, idx_v, ebufs, xbufs,
             gsem, isem, osem):
        wid = lax.axis_index("s") * NUM_CORES + lax.axis_index("c")
        s0 = wid * s_per_w
        pltpu.sync_copy(pos_hbm.at[0, pl.ds(s0, s_per_w)], idx_v)

        def gd(i):
            p = lax.rem(i, PAR)
            return pltpu.make_async_copy(
                table_hbm.at[idx_v.at[pl.ds(i * KC, KC)]],
                ebufs.at[p], gsem.at[p])

        def ind(i, b):
            p = lax.rem(i, PAR)
            return pltpu.make_async_copy(
                x_hbm.at[b, pl.ds(s0 + i * KC, KC), :],
                xbufs.at[p * B + b], isem.at[p])

        def outd(i, b):
            p = lax.rem(i, PAR)
            return pltpu.make_async_copy(
                xbufs.at[p * B + b],
                out_hbm.at[b, pl.ds(s0 + i * KC, KC), :],
                osem.at[p])

        def grouped(i, sem):
            p = lax.rem(i, PAR)
            return pltpu.make_async_copy(
                x_hbm.at[pl.ds(0, B), pl.ds(0, KC), :],
                xbufs.at[pl.ds(p * B, B)], sem.at[p])

        for j in range(PAR - 1):
            gd(j).start()
            for b in range(B):
                ind(j, b).start()

        @pl.loop(0, n_chunks, unroll=2)
        def _(i):
            par = lax.rem(i, PAR)
            gd(i).wait()

            @pl.when(i + PAR - 1 < n_chunks)
            def _():
                gd(i + PAR - 1).start()

            grouped(i, isem).wait()

            @pl.loop(0, KC)
            def _(r):
                for c in range(D // LANES):
                    sl = pl.ds(c * LANES, LANES)
                    v = ebufs[par, r, sl]
                    for b in range(B):
                        plsc.addupdate(xbufs.at[par * B + b, r, sl], v)

            for b in range(B):
                outd(i, b).start()

            @pl.when(i + PAR - 1 < n_chunks)
            def _():
                @pl.when(i >= 1)
                def _():
                    grouped(i - 1, osem).wait()

                for b in range(B):
                    ind(i + PAR - 1, b).start()

        for j in range(max(0, n_chunks - PAR), n_chunks):
            grouped(j, osem).wait()

    return body(x, table, pos2d)


def kernel(x, table, positional_ids):
    return _pos_embed_add(x, table, positional_ids.astype(jnp.int32))


# speedup vs baseline: 1.0031x; 1.0031x over previous
import functools

import jax
import jax.numpy as jnp
from jax import lax
from jax.experimental import pallas as pl
from jax.experimental.pallas import tpu as pltpu
from jax.experimental.pallas import tpu_sc as plsc

NUM_CORES = 2
NUM_SUBCORES = 16
NUM_WORKERS = NUM_CORES * NUM_SUBCORES
LANES = 16
KC = 8
PAR = 3


@jax.jit
def _pos_embed_add(x, table, pos2d):
    B, S, D = x.shape
    s_per_w = S // NUM_WORKERS
    n_chunks = s_per_w // KC

    @functools.partial(
        pl.kernel,
        out_type=jax.ShapeDtypeStruct((B, S, D), jnp.float32),
        mesh=plsc.VectorSubcoreMesh(
            core_axis_name="c", subcore_axis_name="s"
        ),
        scratch_types=[
            pltpu.VMEM((s_per_w,), jnp.int32),
            pltpu.VMEM((PAR, KC, D), jnp.float32),
            pltpu.VMEM((PAR, B, KC, D), jnp.float32),
            pltpu.SemaphoreType.DMA((PAR,)),
            pltpu.SemaphoreType.DMA((PAR,)),
            pltpu.SemaphoreType.DMA((PAR,)),
        ],
    )
    def body(x_hbm, table_hbm, pos_hbm, out_hbm, idx_v, ebufs, xbufs,
             gsem, isem, osem):
        wid = lax.axis_index("s") * NUM_CORES + lax.axis_index("c")
        s0 = wid * s_per_w
        pltpu.sync_copy(pos_hbm.at[0, pl.ds(s0, s_per_w)], idx_v)

        def gd(i):
            p = lax.rem(i, PAR)
            return pltpu.make_async_copy(
                table_hbm.at[idx_v.at[pl.ds(i * KC, KC)]],
                ebufs.at[p], gsem.at[p])

        def ind(i):
            p = lax.rem(i, PAR)
            return pltpu.make_async_copy(
                x_hbm.at[pl.ds(0, B), pl.ds(s0 + i * KC, KC), :],
                xbufs.at[p], isem.at[p])

        def outd(i):
            p = lax.rem(i, PAR)
            return pltpu.make_async_copy(
                xbufs.at[p],
                out_hbm.at[pl.ds(0, B), pl.ds(s0 + i * KC, KC), :],
                osem.at[p])

        for j in range(PAR - 1):
            gd(j).start()
            ind(j).start()

        @pl.loop(0, n_chunks, unroll=2)
        def _(i):
            par = lax.rem(i, PAR)
            gd(i).wait()

            @pl.when(i + PAR - 1 < n_chunks)
            def _():
                gd(i + PAR - 1).start()

            ind(i).wait()

            @pl.loop(0, KC)
            def _(r):
                for c in range(D // LANES):
                    sl = pl.ds(c * LANES, LANES)
                    v = ebufs[par, r, sl]
                    for b in range(B):
                        plsc.addupdate(xbufs.at[par, b, r, sl], v)

            outd(i).start()

            @pl.when(i + PAR - 1 < n_chunks)
            def _():
                @pl.when(i >= 1)
                def _():
                    outd(i - 1).wait()

                ind(i + PAR - 1).start()

        for j in range(max(0, n_chunks - PAR), n_chunks):
            outd(j).wait()

    return body(x, table, pos2d)


def kernel(x, table, positional_ids):
    return _pos_embed_add(x, table, positional_ids.astype(jnp.int32))
